# baseline (speedup 1.0000x reference)
import functools

import jax
import jax.numpy as jnp
from jax import lax
from jax.experimental import pallas as pl
from jax.experimental.pallas import tpu as pltpu
from jax.experimental.pallas import tpu_sc as plsc

_BATCH = 256
_N_TOK = 77
_D = 768
_ROWS = _BATCH * _N_TOK
_CH = 64
_LANES = 16


def _sc_embed(tokens_flat, table, pos):
    info = plsc.get_sparse_core_info()
    num_workers = info.num_cores * info.num_subcores
    rows_per_w = _ROWS // num_workers
    num_chunks = (rows_per_w + _CH - 1) // _CH

    mesh = plsc.VectorSubcoreMesh(core_axis_name="c", subcore_axis_name="s")

    @functools.partial(
        pl.kernel,
        mesh=mesh,
        out_type=jax.ShapeDtypeStruct((_ROWS, _D), jnp.float32),
        scratch_types=[
            pltpu.VMEM((_CH,), jnp.int32),
            pltpu.VMEM((_CH, _D), jnp.float32),
            pltpu.VMEM((_N_TOK, _D), jnp.float32),
            pltpu.SemaphoreType.DMA,
        ],
    )
    def body(tok_hbm, tab_hbm, pos_hbm, out_hbm, idx_v, buf_v, pos_v, sem):
        wid = lax.axis_index("s") * info.num_cores + lax.axis_index("c")
        r0 = wid * rows_per_w
        pltpu.sync_copy(pos_hbm, pos_v)

        def chunk(c, carry):
            s = r0 + jnp.minimum(c * _CH, rows_per_w - _CH)
            pltpu.sync_copy(tok_hbm.at[pl.ds(s, _CH)], idx_v)
            pltpu.async_copy(tab_hbm.at[idx_v], buf_v, sem).wait()

            def row(i, p):
                for k in range(_D // _LANES):
                    sl = pl.ds(k * _LANES, _LANES)
                    buf_v[i, sl] = buf_v[i, sl] + pos_v[p, sl]
                return lax.select(p == _N_TOK - 1, 0, p + 1)

            lax.fori_loop(0, _CH, row, lax.rem(s, _N_TOK))
            pltpu.sync_copy(buf_v, out_hbm.at[pl.ds(s, _CH)])
            return carry

        lax.fori_loop(0, num_chunks, chunk, 0)

    return body(tokens_flat, table, pos)


def kernel(tokens, token_embedding, positional_embedding):
    out = _sc_embed(
        tokens.astype(jnp.int32).reshape(-1),
        token_embedding,
        positional_embedding,
    )
    return out.reshape(_BATCH, _N_TOK, _D)


# speedup vs baseline: 1.0987x; 1.0987x over previous
import functools

import jax
import jax.numpy as jnp
from jax import lax
from jax.experimental import pallas as pl
from jax.experimental.pallas import tpu as pltpu
from jax.experimental.pallas import tpu_sc as plsc

_BATCH = 256
_N_TOK = 77
_D = 768
_ROWS = _BATCH * _N_TOK
_CH = 32
_LANES = 16


def _sc_embed(tokens_flat, table, pos):
    info = plsc.get_sparse_core_info()
    num_workers = info.num_cores * info.num_subcores
    rows_per_w = _ROWS // num_workers
    num_chunks = (rows_per_w + _CH - 1) // _CH
    num_pairs = num_chunks // 2
    last_off = rows_per_w - _CH

    mesh = plsc.VectorSubcoreMesh(core_axis_name="c", subcore_axis_name="s")

    @functools.partial(
        pl.kernel,
        mesh=mesh,
        out_type=jax.ShapeDtypeStruct((_ROWS, _D), jnp.float32),
        scratch_types=[
            pltpu.VMEM((rows_per_w,), jnp.int32),
            pltpu.VMEM((_CH, _D), jnp.float32),
            pltpu.VMEM((_CH, _D), jnp.float32),
            pltpu.VMEM((_N_TOK, _D), jnp.float32),
            pltpu.SemaphoreType.DMA,
            pltpu.SemaphoreType.DMA,
            pltpu.SemaphoreType.DMA,
            pltpu.SemaphoreType.DMA,
        ],
    )
    def body(
        tok_hbm, tab_hbm, pos_hbm, out_hbm,
        idx_v, buf0, buf1, pos_v, gsem0, gsem1, wsem0, wsem1,
    ):
        wid = lax.axis_index("s") * info.num_cores + lax.axis_index("c")
        r0 = wid * rows_per_w
        pltpu.sync_copy(tok_hbm.at[pl.ds(r0, rows_per_w)], idx_v)
        pltpu.sync_copy(pos_hbm, pos_v)

        def off(c):
            return jnp.minimum(c * _CH, last_off)

        def start_gather(c, buf, sem):
            pltpu.async_copy(tab_hbm.at[idx_v.at[pl.ds(off(c), _CH)]], buf, sem)

        def wait(buf, sem):
            pltpu.make_async_copy(tab_hbm.at[idx_v.at[pl.ds(0, _CH)]], buf, sem).wait()

        def start_wb(c, buf, sem):
            pltpu.async_copy(buf, out_hbm.at[pl.ds(r0 + off(c), _CH)], sem)

        def wait_wb(c, buf, sem):
            pltpu.make_async_copy(buf, out_hbm.at[pl.ds(r0 + off(c), _CH)], sem).wait()

        def add_pos(c, buf):
            p0 = lax.rem(off(c), _N_TOK)

            def row(i, carry):
                p = lax.rem(p0 + i, _N_TOK)
                for k in range(_D // _LANES):
                    sl = pl.ds(k * _LANES, _LANES)
                    buf[i, sl] = buf[i, sl] + pos_v[p, sl]
                return carry

            lax.fori_loop(0, _CH, row, 0)

        start_gather(0, buf0, gsem0)

        def pair(t, carry):
            e = 2 * t
            o = e + 1

            @pl.when(t > 0)
            def _():
                wait_wb(o - 2, buf1, wsem1)

            start_gather(o, buf1, gsem1)
            wait(buf0, gsem0)
            add_pos(e, buf0)
            start_wb(e, buf0, wsem0)
            wait(buf1, gsem1)
            add_pos(o, buf1)
            start_wb(o, buf1, wsem1)
            wait_wb(e, buf0, wsem0)

            @pl.when(t < num_pairs - 1)
            def _():
                start_gather(e + 2, buf0, gsem0)

            return carry

        lax.fori_loop(0, num_pairs, pair, 0)
        wait_wb(num_chunks - 1, buf1, wsem1)

    return body(tokens_flat, table, pos)


def kernel(tokens, token_embedding, positional_embedding):
    out = _sc_embed(
        tokens.astype(jnp.int32).reshape(-1),
        token_embedding,
        positional_embedding,
    )
    return out.reshape(_BATCH, _N_TOK, _D)


# speedup vs baseline: 1.7011x; 1.5483x over previous
import functools

import jax
import jax.numpy as jnp
from jax import lax
from jax.experimental import pallas as pl
from jax.experimental.pallas import tpu as pltpu
from jax.experimental.pallas import tpu_sc as plsc

_BATCH = 256
_N_TOK = 77
_D = 768
_ROWS = _BATCH * _N_TOK
_CH = 32
_LANES = 16


def _sc_embed(tokens_flat, table, pos):
    info = plsc.get_sparse_core_info()
    num_workers = info.num_cores * info.num_subcores
    rows_per_w = _ROWS // num_workers
    num_chunks = (rows_per_w + _CH - 1) // _CH
    num_pairs = num_chunks // 2
    last_off = rows_per_w - _CH

    mesh = plsc.VectorSubcoreMesh(core_axis_name="c", subcore_axis_name="s")

    @functools.partial(
        pl.kernel,
        mesh=mesh,
        out_type=jax.ShapeDtypeStruct((_ROWS, _D), jnp.float32),
        scratch_types=[
            pltpu.VMEM((rows_per_w,), jnp.int32),
            pltpu.VMEM((_CH, _D), jnp.float32),
            pltpu.VMEM((_CH, _D), jnp.float32),
            pltpu.VMEM((_N_TOK, _D), jnp.float32),
            pltpu.SemaphoreType.DMA,
            pltpu.SemaphoreType.DMA,
            pltpu.SemaphoreType.DMA,
            pltpu.SemaphoreType.DMA,
        ],
    )
    def body(
        tok_hbm, tab_hbm, pos_hbm, out_hbm,
        idx_v, buf0, buf1, pos_v, gsem0, gsem1, wsem0, wsem1,
    ):
        wid = lax.axis_index("s") * info.num_cores + lax.axis_index("c")
        r0 = wid * rows_per_w
        pltpu.sync_copy(tok_hbm.at[pl.ds(r0, rows_per_w)], idx_v)
        pltpu.sync_copy(pos_hbm, pos_v)

        def off(c):
            return jnp.minimum(c * _CH, last_off)

        def start_gather(c, buf, sem):
            pltpu.async_copy(tab_hbm.at[idx_v.at[pl.ds(off(c), _CH)]], buf, sem)

        def wait(buf, sem):
            pltpu.make_async_copy(tab_hbm.at[idx_v.at[pl.ds(0, _CH)]], buf, sem).wait()

        def start_wb(c, buf, sem):
            pltpu.async_copy(buf, out_hbm.at[pl.ds(r0 + off(c), _CH)], sem)

        def wait_wb(c, buf, sem):
            pltpu.make_async_copy(buf, out_hbm.at[pl.ds(r0 + off(c), _CH)], sem).wait()

        def add_pos(c, buf):
            p0 = lax.rem(off(c), _N_TOK)

            @plsc.parallel_loop(0, _CH)
            def _(i):
                p = lax.rem(p0 + i, _N_TOK)

                @plsc.parallel_loop(0, _D // _LANES, unroll=8)
                def _(k):
                    sl = pl.ds(k * _LANES, _LANES)
                    buf[i, sl] = buf[i, sl] + pos_v[p, sl]

        start_gather(0, buf0, gsem0)

        def pair(t, carry):
            e = 2 * t
            o = e + 1

            @pl.when(t > 0)
            def _():
                wait_wb(o - 2, buf1, wsem1)

            start_gather(o, buf1, gsem1)
            wait(buf0, gsem0)
            add_pos(e, buf0)
            start_wb(e, buf0, wsem0)
            wait(buf1, gsem1)
            add_pos(o, buf1)
            start_wb(o, buf1, wsem1)
            wait_wb(e, buf0, wsem0)

            @pl.when(t < num_pairs - 1)
            def _():
                start_gather(e + 2, buf0, gsem0)

            return carry

        lax.fori_loop(0, num_pairs, pair, 0)
        wait_wb(num_chunks - 1, buf1, wsem1)

    return body(tokens_flat, table, pos)


def kernel(tokens, token_embedding, positional_embedding):
    out = _sc_embed(
        tokens.astype(jnp.int32).reshape(-1),
        token_embedding,
        positional_embedding,
    )
    return out.reshape(_BATCH, _N_TOK, _D)
